# baseline (speedup 1.0000x reference)
import functools

import jax
import jax.numpy as jnp
from jax import lax
from jax.experimental import pallas as pl
from jax.experimental.pallas import tpu as pltpu
from jax.experimental.pallas import tpu_sc as plsc

_B, _S, _C = 4096, 26, 1000
_ROWS = _B * _S
_NW = 32
_RPW = _ROWS // _NW
_CHUNK = 32
_NCH = _RPW // _CHUNK
_NPAIRS = _NCH // 2
_BUFW = _CHUNK * _C


def _scatter_val(buf, idx_v, chunk, val):
    vals = jnp.full((16,), val, jnp.float32)
    lane = lax.iota(jnp.int32, 16)
    for g in range(_CHUNK // 16):
        cols = idx_v[pl.ds(chunk * _CHUNK + g * 16, 16)]
        flat = (lane + g * 16) * _C + cols
        plsc.store_scatter(buf, [flat], vals)


def _body(idx_hbm, out_hbm, idx_v, buf0, buf1, sem0, sem1):
    wid = lax.axis_index("s") * 2 + lax.axis_index("c")
    base = wid * _RPW
    pltpu.sync_copy(idx_hbm.at[pl.ds(base, _RPW)], idx_v)

    bufs = (buf0, buf1)
    sems = (sem0, sem1)
    zeros = jnp.zeros((16,), jnp.float32)

    def zero_body(i, _):
        buf0[pl.ds(i * 16, 16)] = zeros
        buf1[pl.ds(i * 16, 16)] = zeros
        return 0

    lax.fori_loop(0, _BUFW // 16, zero_body, 0)

    def out_slice(chunk):
        return out_hbm.at[pl.ds((base + chunk * _CHUNK) * _C, _BUFW)]

    for b in range(2):
        _scatter_val(bufs[b], idx_v, b, 1.0)
        pltpu.async_copy(bufs[b], out_slice(b), sems[b])

    def pair_body(p, _):
        for b in range(2):
            cur = p * 2 + b
            prev = cur - 2
            pltpu.make_async_copy(bufs[b], out_slice(prev), sems[b]).wait()
            _scatter_val(bufs[b], idx_v, prev, 0.0)
            _scatter_val(bufs[b], idx_v, cur, 1.0)
            pltpu.async_copy(bufs[b], out_slice(cur), sems[b])
        return 0

    lax.fori_loop(1, _NPAIRS, pair_body, 0)

    for b in range(2):
        last = (_NPAIRS - 1) * 2 + b
        pltpu.make_async_copy(bufs[b], out_slice(last), sems[b]).wait()


@jax.jit
def _onehot_flat(idx_flat):
    mesh = plsc.VectorSubcoreMesh(core_axis_name="c", subcore_axis_name="s")
    return pl.kernel(
        _body,
        out_type=jax.ShapeDtypeStruct((_ROWS * _C,), jnp.float32),
        mesh=mesh,
        compiler_params=pltpu.CompilerParams(needs_layout_passes=False),
        scratch_types=[
            pltpu.VMEM((_RPW,), jnp.int32),
            pltpu.VMEM((_BUFW,), jnp.float32),
            pltpu.VMEM((_BUFW,), jnp.float32),
            pltpu.SemaphoreType.DMA,
            pltpu.SemaphoreType.DMA,
        ],
    )(idx_flat)


def kernel(inputs):
    idx_flat = inputs.reshape(-1).astype(jnp.int32)
    return _onehot_flat(idx_flat).reshape(_B, _S, _C)


# speedup vs baseline: 1.9704x; 1.9704x over previous
import functools

import jax
import jax.numpy as jnp
from jax import lax
from jax.experimental import pallas as pl
from jax.experimental.pallas import tpu as pltpu
from jax.experimental.pallas import tpu_sc as plsc

_B, _S, _C = 4096, 26, 1000
_NW = 32
_BPW = _B // _NW
_IPW = _BPW * _S
_IDXPAD = 3344


def _scatter_val(buf, idx_v, lb, vals, mask10):
    lane = lax.iota(jnp.int32, 16)
    cols0 = plsc.load_gather(idx_v, [lb * _S + lane])
    plsc.store_scatter(buf, [lane, cols0], vals)
    cols1 = plsc.load_gather(idx_v, [lb * _S + 16 + lane])
    plsc.store_scatter(buf, [lane + 16, cols1], vals, mask=mask10)


def _body(idx_hbm, out_hbm, idx_v, buf0, buf1, sem0, sem1):
    wid = lax.axis_index("s") * 2 + lax.axis_index("c")
    base_b = wid * _BPW
    pltpu.sync_copy(idx_hbm.at[pl.ds(wid * _IPW, _IPW)],
                    idx_v.at[pl.ds(0, _IPW)])

    bufs = (buf0, buf1)
    sems = (sem0, sem1)
    lane = lax.iota(jnp.int32, 16)
    mask10 = lane < 10
    ones = jnp.full((16,), 1.0, jnp.float32)
    zeros = jnp.zeros((16,), jnp.float32)

    def zero_body(i, _):
        for r in range(2):
            for c in range(_C // 8 - 1):
                bufs[r][i, pl.ds(c * 8, 16)] = zeros
        return 0

    lax.fori_loop(0, _S, zero_body, 0)

    for b in range(2):
        _scatter_val(bufs[b], idx_v, b, ones, mask10)
        pltpu.async_copy(bufs[b], out_hbm.at[base_b + b], sems[b])

    def pair_body(p, _):
        for b in range(2):
            cur = p * 2 + b
            prev = cur - 2
            pltpu.make_async_copy(
                bufs[b], out_hbm.at[base_b + prev], sems[b]).wait()
            _scatter_val(bufs[b], idx_v, prev, zeros, mask10)
            _scatter_val(bufs[b], idx_v, cur, ones, mask10)
            pltpu.async_copy(bufs[b], out_hbm.at[base_b + cur], sems[b])
        return 0

    lax.fori_loop(1, _BPW // 2, pair_body, 0)

    for b in range(2):
        last = _BPW - 2 + b
        pltpu.make_async_copy(
            bufs[b], out_hbm.at[base_b + last], sems[b]).wait()


@jax.jit
def _onehot(idx_flat):
    mesh = plsc.VectorSubcoreMesh(core_axis_name="c", subcore_axis_name="s")
    return pl.kernel(
        _body,
        out_type=jax.ShapeDtypeStruct((_B, _S, _C), jnp.float32),
        mesh=mesh,
        compiler_params=pltpu.CompilerParams(needs_layout_passes=False),
        scratch_types=[
            pltpu.VMEM((_IDXPAD,), jnp.int32),
            pltpu.VMEM((_S, _C), jnp.float32),
            pltpu.VMEM((_S, _C), jnp.float32),
            pltpu.SemaphoreType.DMA,
            pltpu.SemaphoreType.DMA,
        ],
    )(idx_flat)


def kernel(inputs):
    idx_flat = inputs.reshape(-1).astype(jnp.int32)
    return _onehot(idx_flat)


# speedup vs baseline: 8.2095x; 4.1665x over previous
import functools

import jax
import jax.numpy as jnp
from jax import lax
from jax.experimental import pallas as pl
from jax.experimental.pallas import tpu as pltpu
from jax.experimental.pallas import tpu_sc as plsc

_B, _S, _C = 4096, 26, 1000
_NW = 32
_BPW = _B // _NW
_IPW = _BPW * _S
_CW = 200
_NCW = _C // _CW
_NSTEP = _S * _NCW


def _scatter_val(buf, idx_v, step, vals):
    s = step // _NCW
    c0 = (step % _NCW) * _CW
    lane = lax.iota(jnp.int32, 16)
    for g in range(_BPW // 16):
        blane = g * 16 + lane
        cols = plsc.load_gather(idx_v, [blane * _S + s])
        rel = cols - c0
        m = (rel >= 0) & (rel < _CW)
        plsc.store_scatter(buf, [rel, blane], vals, mask=m)


def _body(idx_hbm, out_hbm, idx_v, buf0, buf1, sem0, sem1):
    wid = lax.axis_index("s") * 2 + lax.axis_index("c")
    b0 = wid * _BPW
    pltpu.sync_copy(idx_hbm.at[pl.ds(b0 * _S, _IPW)], idx_v)

    bufs = (buf0, buf1)
    sems = (sem0, sem1)
    zeros16 = jnp.zeros((16,), jnp.float32)
    ones = jnp.full((16,), 1.0, jnp.float32)
    zeros = jnp.zeros((16,), jnp.float32)

    def zero_body(i, _):
        for r in range(2):
            for c in range(_BPW // 16):
                bufs[r][i, pl.ds(c * 16, 16)] = zeros16
        return 0

    lax.fori_loop(0, _CW, zero_body, 0)

    def out_slice(step):
        s = step // _NCW
        c0 = (step % _NCW) * _CW
        return out_hbm.at[s, pl.ds(c0, _CW), pl.ds(b0, _BPW)]

    for b in range(2):
        _scatter_val(bufs[b], idx_v, b, ones)
        pltpu.async_copy(bufs[b], out_slice(b), sems[b])

    def pair_body(p, _):
        for b in range(2):
            cur = p * 2 + b
            prev = cur - 2
            pltpu.make_async_copy(bufs[b], out_slice(prev), sems[b]).wait()
            _scatter_val(bufs[b], idx_v, prev, zeros)
            _scatter_val(bufs[b], idx_v, cur, ones)
            pltpu.async_copy(bufs[b], out_slice(cur), sems[b])
        return 0

    lax.fori_loop(1, _NSTEP // 2, pair_body, 0)

    for b in range(2):
        last = _NSTEP - 2 + b
        pltpu.make_async_copy(bufs[b], out_slice(last), sems[b]).wait()


@jax.jit
def _onehot(idx_flat):
    mesh = plsc.VectorSubcoreMesh(core_axis_name="c", subcore_axis_name="s")
    res = pl.kernel(
        _body,
        out_type=jax.ShapeDtypeStruct((_S, _C, _B), jnp.float32),
        mesh=mesh,
        compiler_params=pltpu.CompilerParams(needs_layout_passes=False),
        scratch_types=[
            pltpu.VMEM((_IPW,), jnp.int32),
            pltpu.VMEM((_CW, _BPW), jnp.float32),
            pltpu.VMEM((_CW, _BPW), jnp.float32),
            pltpu.SemaphoreType.DMA,
            pltpu.SemaphoreType.DMA,
        ],
    )(idx_flat)
    return jnp.transpose(res, (2, 0, 1))


def kernel(inputs):
    idx_flat = inputs.reshape(-1).astype(jnp.int32)
    return _onehot(idx_flat)


# speedup vs baseline: 8.2825x; 1.0089x over previous
import functools

import jax
import jax.numpy as jnp
from jax import lax
from jax.experimental import pallas as pl
from jax.experimental.pallas import tpu as pltpu
from jax.experimental.pallas import tpu_sc as plsc

_B, _S, _C = 4096, 26, 1000
_NW = 32
_BPW = _B // _NW
_IPW = _BPW * _S
_CW = 200
_NCW = _C // _CW
_NSTEP = _S * _NCW


def _scatter_val(buf, idx_v, step, vals):
    s = step // _NCW
    c0 = (step % _NCW) * _CW
    lane = lax.iota(jnp.int32, 16)
    svec = jnp.full((16,), s, jnp.int32)
    for g in range(_BPW // 16):
        blane = g * 16 + lane
        cols = plsc.load_gather(idx_v, [svec, blane])
        rel = cols - c0
        m = (rel >= 0) & (rel < _CW)
        plsc.store_scatter(buf, [rel, blane], vals, mask=m)


def _body(idx_hbm, out_hbm, idx_v, buf0, buf1, sem0, sem1):
    wid = lax.axis_index("s") * 2 + lax.axis_index("c")
    b0 = wid * _BPW
    pltpu.sync_copy(idx_hbm.at[:, pl.ds(b0, _BPW)], idx_v)

    bufs = (buf0, buf1)
    sems = (sem0, sem1)
    zeros16 = jnp.zeros((16,), jnp.float32)
    ones = jnp.full((16,), 1.0, jnp.float32)
    zeros = jnp.zeros((16,), jnp.float32)

    def make_zero_body(r):
        def zero_body(i, _):
            for c in range(_BPW // 16):
                bufs[r][i, pl.ds(c * 16, 16)] = zeros16
            return 0
        return zero_body

    def out_slice(step):
        s = step // _NCW
        c0 = (step % _NCW) * _CW
        return out_hbm.at[s, pl.ds(c0, _CW), pl.ds(b0, _BPW)]

    for b in range(2):
        lax.fori_loop(0, _CW, make_zero_body(b), 0)
        _scatter_val(bufs[b], idx_v, b, ones)
        pltpu.async_copy(bufs[b], out_slice(b), sems[b])

    def pair_body(p, _):
        for b in range(2):
            cur = p * 2 + b
            prev = cur - 2
            pltpu.make_async_copy(bufs[b], out_slice(prev), sems[b]).wait()
            _scatter_val(bufs[b], idx_v, prev, zeros)
            _scatter_val(bufs[b], idx_v, cur, ones)
            pltpu.async_copy(bufs[b], out_slice(cur), sems[b])
        return 0

    lax.fori_loop(1, _NSTEP // 2, pair_body, 0)

    for b in range(2):
        last = _NSTEP - 2 + b
        pltpu.make_async_copy(bufs[b], out_slice(last), sems[b]).wait()


@jax.jit
def _onehot(idx2d):
    mesh = plsc.VectorSubcoreMesh(core_axis_name="c", subcore_axis_name="s")
    res = pl.kernel(
        _body,
        out_type=jax.ShapeDtypeStruct((_S, _C, _B), jnp.float32),
        mesh=mesh,
        compiler_params=pltpu.CompilerParams(needs_layout_passes=False),
        scratch_types=[
            pltpu.VMEM((_S, _BPW), jnp.int32),
            pltpu.VMEM((_CW, _BPW), jnp.float32),
            pltpu.VMEM((_CW, _BPW), jnp.float32),
            pltpu.SemaphoreType.DMA,
            pltpu.SemaphoreType.DMA,
        ],
    )(idx2d)
    return jnp.transpose(res, (2, 0, 1))


def kernel(inputs):
    return _onehot(inputs.astype(jnp.int32).T)


# speedup vs baseline: 8.2917x; 1.0011x over previous
import functools

import jax
import jax.numpy as jnp
from jax import lax
from jax.experimental import pallas as pl
from jax.experimental.pallas import tpu as pltpu
from jax.experimental.pallas import tpu_sc as plsc

_B, _S, _C = 4096, 26, 1000
_NW = 32
_BPW = _B // _NW
_IPW = _BPW * _S
_CW = 200
_NCW = _C // _CW
_NSTEP = _S * _NCW


def _scatter_val(buf, idx_v, step, vals):
    s = step // _NCW
    c0 = (step % _NCW) * _CW
    lane = lax.iota(jnp.int32, 16)
    svec = jnp.full((16,), s, jnp.int32)
    for g in range(_BPW // 16):
        blane = g * 16 + lane
        cols = plsc.load_gather(idx_v, [svec, blane])
        rel = cols - c0
        m = (rel >= 0) & (rel < _CW)
        plsc.store_scatter(buf, [rel, blane], vals, mask=m)


def _body(idx_hbm, out_hbm, idx_v, buf0, buf1, sem0, sem1):
    wid = lax.axis_index("s") * 2 + lax.axis_index("c")
    b0 = wid * _BPW
    pltpu.sync_copy(idx_hbm.at[:, pl.ds(b0, _BPW)], idx_v)

    bufs = (buf0, buf1)
    sems = (sem0, sem1)
    zeros16 = jnp.zeros((16,), jnp.float32)
    ones = jnp.full((16,), 1.0, jnp.float32)
    zeros = jnp.zeros((16,), jnp.float32)

    def make_zero_body(r):
        def zero_body(i, _):
            for c in range(_BPW // 16):
                bufs[r][i, pl.ds(c * 16, 16)] = zeros16
            return 0
        return zero_body

    def out_slice(step):
        s = step // _NCW
        c0 = (step % _NCW) * _CW
        return out_hbm.at[s, pl.ds(c0, _CW), pl.ds(b0, _BPW)]

    for b in range(2):
        lax.fori_loop(0, _CW, make_zero_body(b), 0)
        _scatter_val(bufs[b], idx_v, b, ones)
        pltpu.async_copy(bufs[b], out_slice(b), sems[b])

    def pair_body(p, _):
        for b in range(2):
            cur = p * 2 + b
            prev = cur - 2
            pltpu.make_async_copy(bufs[b], out_slice(prev), sems[b]).wait()
            _scatter_val(bufs[b], idx_v, prev, zeros)
            _scatter_val(bufs[b], idx_v, cur, ones)
            pltpu.async_copy(bufs[b], out_slice(cur), sems[b])
        return 0

    lax.fori_loop(1, _NSTEP // 2, pair_body, 0)

    for b in range(2):
        last = _NSTEP - 2 + b
        pltpu.make_async_copy(bufs[b], out_slice(last), sems[b]).wait()


@jax.jit
def _onehot(idx2d):
    mesh = plsc.VectorSubcoreMesh(core_axis_name="c", subcore_axis_name="s")
    res = pl.kernel(
        _body,
        out_type=jax.ShapeDtypeStruct((_S, _C, _B), jnp.float32),
        mesh=mesh,
        compiler_params=pltpu.CompilerParams(
            needs_layout_passes=False, skip_device_barrier=True),
        scratch_types=[
            pltpu.VMEM((_S, _BPW), jnp.int32),
            pltpu.VMEM((_CW, _BPW), jnp.float32),
            pltpu.VMEM((_CW, _BPW), jnp.float32),
            pltpu.SemaphoreType.DMA,
            pltpu.SemaphoreType.DMA,
        ],
    )(idx2d)
    return jnp.transpose(res, (2, 0, 1))


def kernel(inputs):
    return _onehot(inputs.astype(jnp.int32).T)


# speedup vs baseline: 8.4069x; 1.0139x over previous
import functools

import jax
import jax.numpy as jnp
from jax import lax
from jax.experimental import pallas as pl
from jax.experimental.pallas import tpu as pltpu
from jax.experimental.pallas import tpu_sc as plsc

_B, _S, _C = 4096, 26, 1000
_NW = 32
_BPW = _B // _NW
_IPW = _BPW * _S
_CW = 200
_NCW = _C // _CW
_STEC = 24
_NSTEP = _STEC * _NCW


def _scatter_val(buf, idx_v, step, vals):
    s = step // _NCW
    c0 = (step % _NCW) * _CW
    lane = lax.iota(jnp.int32, 16)
    svec = jnp.full((16,), s, jnp.int32)
    for g in range(_BPW // 16):
        blane = g * 16 + lane
        cols = plsc.load_gather(idx_v, [svec, blane])
        rel = cols - c0
        m = (rel >= 0) & (rel < _CW)
        plsc.store_scatter(buf, [rel, blane], vals, mask=m)


def _body(idx_hbm, out_hbm, idx_v, buf0, buf1, sem0, sem1):
    wid = lax.axis_index("s") * 2 + lax.axis_index("c")
    b0 = wid * _BPW
    pltpu.sync_copy(idx_hbm.at[:, pl.ds(b0, _BPW)], idx_v)

    bufs = (buf0, buf1)
    sems = (sem0, sem1)
    zeros16 = jnp.zeros((16,), jnp.float32)
    ones = jnp.full((16,), 1.0, jnp.float32)
    zeros = jnp.zeros((16,), jnp.float32)

    def make_zero_body(r):
        def zero_body(i, _):
            for c in range(_BPW // 16):
                bufs[r][i, pl.ds(c * 16, 16)] = zeros16
            return 0
        return zero_body

    def out_slice(step):
        s = step // _NCW
        c0 = (step % _NCW) * _CW
        return out_hbm.at[s, pl.ds(c0, _CW), pl.ds(b0, _BPW)]

    for b in range(2):
        lax.fori_loop(0, _CW, make_zero_body(b), 0)
        _scatter_val(bufs[b], idx_v, b, ones)
        pltpu.async_copy(bufs[b], out_slice(b), sems[b])

    def pair_body(p, _):
        for b in range(2):
            cur = p * 2 + b
            prev = cur - 2
            pltpu.make_async_copy(bufs[b], out_slice(prev), sems[b]).wait()
            _scatter_val(bufs[b], idx_v, prev, zeros)
            _scatter_val(bufs[b], idx_v, cur, ones)
            pltpu.async_copy(bufs[b], out_slice(cur), sems[b])
        return 0

    lax.fori_loop(1, _NSTEP // 2, pair_body, 0)

    for b in range(2):
        last = _NSTEP - 2 + b
        pltpu.make_async_copy(bufs[b], out_slice(last), sems[b]).wait()


def _scs_body(idx_hbm, out_hbm, zbuf, zsem):
    del idx_hbm
    cid = lax.axis_index("c")
    s = _STEC + cid
    for k in range(_B // 512):
        pltpu.async_copy(zbuf, out_hbm.at[s, :, pl.ds(k * 512, 512)], zsem)
    for k in range(_B // 512):
        pltpu.make_async_copy(
            zbuf, out_hbm.at[s, :, pl.ds(k * 512, 512)], zsem).wait()


def _tec_scoped(idx_hbm, out_hbm, idx_v, buf0, buf1, sem0, sem1):
    _body(idx_hbm, out_hbm, idx_v, buf0, buf1, sem0, sem1)


def _tec_body(idx_hbm, out_hbm, zbuf, zsem):
    del zbuf, zsem
    pl.run_scoped(
        functools.partial(_tec_scoped, idx_hbm, out_hbm),
        pltpu.VMEM((_S, _BPW), jnp.int32),
        pltpu.VMEM((_CW, _BPW), jnp.float32),
        pltpu.VMEM((_CW, _BPW), jnp.float32),
        pltpu.SemaphoreType.DMA,
        pltpu.SemaphoreType.DMA,
    )


@jax.jit
def _onehot(idx2d):
    smesh = plsc.ScalarSubcoreMesh(axis_name="c")
    vmesh = plsc.VectorSubcoreMesh(core_axis_name="c", subcore_axis_name="s")
    res = pl.kernel(
        [_scs_body, _tec_body],
        out_type=pltpu.MemorySpace.HBM((_S, _C, _B), jnp.float32),
        mesh=[smesh, vmesh],
        compiler_params=pltpu.CompilerParams(
            needs_layout_passes=False, skip_device_barrier=True),
        scratch_types=[
            pltpu.MemorySpace.VMEM_SHARED((_C, 512), jnp.float32),
            pltpu.SemaphoreType.DMA @ plsc.ScalarSubcoreMesh(axis_name="c"),
        ],
    )(idx2d)
    return jnp.transpose(res, (2, 0, 1))


def kernel(inputs):
    return _onehot(inputs.astype(jnp.int32).T)
